# baseline (speedup 1.0000x reference)
import functools

import jax
import jax.numpy as jnp
from jax import lax
from jax.experimental import pallas as pl
from jax.experimental.pallas import tpu as pltpu
from jax.experimental.pallas import tpu_sc as plsc

N = 10000
E = 320000
D = 128

NC = 2
NS = 16
NW = NC * NS

NPAD = 10240
STRIPE = NPAD // NS

CHUNK = 80
EPT = E // NW
EPS = E // NS

_MESH = plsc.VectorSubcoreMesh(
    core_axis_name="c", subcore_axis_name="s", num_cores=NC, num_subcores=NS)


def _mm(x, w, c, br=2000):
    r, k = x.shape
    dout = w.shape[1]

    def body(x_ref, w_ref, c_ref, o_ref):
        o_ref[...] = (
            jnp.dot(x_ref[...], w_ref[...], preferred_element_type=jnp.float32)
            + c_ref[...])

    return pl.pallas_call(
        body,
        grid=(r // br,),
        in_specs=[
            pl.BlockSpec((br, k), lambda i: (i, 0)),
            pl.BlockSpec((k, dout), lambda i: (0, 0)),
            pl.BlockSpec((1, dout), lambda i: (0, 0)),
        ],
        out_specs=pl.BlockSpec((br, dout), lambda i: (i, 0)),
        out_shape=jax.ShapeDtypeStruct((r, dout), jnp.float32),
    )(x, w, c)


def _ln_relu(x, s, b):
    m = jnp.mean(x, axis=-1, keepdims=True)
    xc = x - m
    v = jnp.mean(xc * xc, axis=-1, keepdims=True)
    return jax.nn.relu(xc * lax.rsqrt(v + 1e-6) * s + b)


def _edge_finish(m_arr, ga, gb, s, b, br=2000):
    def body(m_ref, a_ref, b_ref, s_ref, bb_ref, o_ref):
        x = m_ref[...] + a_ref[...] + b_ref[...]
        o_ref[...] = _ln_relu(x, s_ref[...], bb_ref[...])

    return pl.pallas_call(
        body,
        grid=(E // br,),
        in_specs=[
            pl.BlockSpec((br, D), lambda i: (i, 0)),
            pl.BlockSpec((br, D), lambda i: (i, 0)),
            pl.BlockSpec((br, D), lambda i: (i, 0)),
            pl.BlockSpec((1, D), lambda i: (0, 0)),
            pl.BlockSpec((1, D), lambda i: (0, 0)),
        ],
        out_specs=pl.BlockSpec((br, D), lambda i: (i, 0)),
        out_shape=jax.ShapeDtypeStruct((E, D), jnp.float32),
    )(m_arr, ga, gb, s, b)


def _node_pre(h_n, w_a, w_b, w_t):
    def body(x_ref, wa_ref, wb_ref, wt_ref, a_ref, b_ref, t_ref):
        x = x_ref[...]
        a_ref[...] = jnp.dot(x, wa_ref[...], preferred_element_type=jnp.float32)
        b_ref[...] = jnp.dot(x, wb_ref[...], preferred_element_type=jnp.float32)
        t_ref[...] = jnp.dot(x, wt_ref[...], preferred_element_type=jnp.float32)

    br = 2000
    sds = jax.ShapeDtypeStruct((N, D), jnp.float32)
    return pl.pallas_call(
        body,
        grid=(N // br,),
        in_specs=[
            pl.BlockSpec((br, D), lambda i: (i, 0)),
            pl.BlockSpec((D, D), lambda i: (0, 0)),
            pl.BlockSpec((D, D), lambda i: (0, 0)),
            pl.BlockSpec((D, D), lambda i: (0, 0)),
        ],
        out_specs=[
            pl.BlockSpec((br, D), lambda i: (i, 0)),
            pl.BlockSpec((br, D), lambda i: (i, 0)),
            pl.BlockSpec((br, D), lambda i: (i, 0)),
        ],
        out_shape=[sds, sds, sds],
    )(h_n, w_a, w_b, w_t)


def _node_update(t, sent, recv, w_s, w_r, c, s, b):
    br = 2000

    def body(t_ref, sp_ref, rp_ref, ws_ref, wr_ref, c_ref, s_ref, b_ref,
             o_ref, nsum_ref, esum_ref):
        i = pl.program_id(0)
        sent_blk = sp_ref[...]
        recv_blk = rp_ref[...]
        x = (t_ref[...]
             + jnp.dot(sent_blk, ws_ref[...], preferred_element_type=jnp.float32)
             + jnp.dot(recv_blk, wr_ref[...], preferred_element_type=jnp.float32)
             + c_ref[...])
        h = _ln_relu(x, s_ref[...], b_ref[...])
        o_ref[...] = h

        @pl.when(i == 0)
        def _():
            nsum_ref[...] = jnp.zeros_like(nsum_ref)
            esum_ref[...] = jnp.zeros_like(esum_ref)

        nsum_ref[...] += jnp.sum(h, axis=0, keepdims=True)
        esum_ref[...] += jnp.sum(sent_blk, axis=0, keepdims=True)

    one = jax.ShapeDtypeStruct((1, D), jnp.float32)
    return pl.pallas_call(
        body,
        grid=(N // br,),
        in_specs=[
            pl.BlockSpec((br, D), lambda i: (i, 0)),
            pl.BlockSpec((br, D), lambda i: (i, 0)),
            pl.BlockSpec((br, D), lambda i: (i, 0)),
            pl.BlockSpec((D, D), lambda i: (0, 0)),
            pl.BlockSpec((D, D), lambda i: (0, 0)),
            pl.BlockSpec((1, D), lambda i: (0, 0)),
            pl.BlockSpec((1, D), lambda i: (0, 0)),
            pl.BlockSpec((1, D), lambda i: (0, 0)),
        ],
        out_specs=[
            pl.BlockSpec((br, D), lambda i: (i, 0)),
            pl.BlockSpec((1, D), lambda i: (0, 0)),
            pl.BlockSpec((1, D), lambda i: (0, 0)),
        ],
        out_shape=[jax.ShapeDtypeStruct((N, D), jnp.float32), one, one],
    )(t, sent, recv, w_s, w_r, c, s, b)


def _global_update(nsum, esum, g, wg, bg, lns, lnb, w_e_g, be, w_n_g, bn):
    def body(ns_ref, es_ref, g_ref, wg_ref, bg_ref, s_ref, b_ref,
             weg_ref, be_ref, wng_ref, bn_ref, g_out, ce_out, cn_out):
        wg = wg_ref[...]
        x = (jnp.dot(ns_ref[...], wg[0:D, :], preferred_element_type=jnp.float32)
             + jnp.dot(es_ref[...], wg[D:2 * D, :], preferred_element_type=jnp.float32)
             + jnp.dot(g_ref[...], wg[2 * D:3 * D, :], preferred_element_type=jnp.float32)
             + bg_ref[...])
        gn = _ln_relu(x, s_ref[...], b_ref[...])
        g_out[...] = gn
        ce_out[...] = jnp.dot(gn, weg_ref[...], preferred_element_type=jnp.float32) + be_ref[...]
        cn_out[...] = jnp.dot(gn, wng_ref[...], preferred_element_type=jnp.float32) + bn_ref[...]

    one = jax.ShapeDtypeStruct((1, D), jnp.float32)
    return pl.pallas_call(
        body,
        out_shape=[one, one, one],
    )(nsum, esum, g, wg, bg, lns, lnb, w_e_g, be, w_n_g, bn)


def _global_final(nsum, esum, g, wg, bg, lns, lnb, wdec, bdec):
    def body(ns_ref, es_ref, g_ref, wg_ref, bg_ref, s_ref, b_ref,
             wd_ref, bd_ref, o_ref):
        wg = wg_ref[...]
        x = (jnp.dot(ns_ref[...], wg[0:D, :], preferred_element_type=jnp.float32)
             + jnp.dot(es_ref[...], wg[D:2 * D, :], preferred_element_type=jnp.float32)
             + jnp.dot(g_ref[...], wg[2 * D:3 * D, :], preferred_element_type=jnp.float32)
             + bg_ref[...])
        gn = _ln_relu(x, s_ref[...], b_ref[...])
        o_ref[...] = jnp.dot(gn, wd_ref[...], preferred_element_type=jnp.float32) + bd_ref[...]

    return pl.pallas_call(
        body,
        out_shape=jax.ShapeDtypeStruct((1, D), jnp.float32),
    )(nsum, esum, g, wg, bg, lns, lnb, wdec, bdec)


def _rsqrt16(x):
    i = lax.bitcast_convert_type(x, jnp.int32)
    i = jnp.int32(0x5F3759DF) - lax.shift_right_arithmetic(i, jnp.int32(1))
    y = lax.bitcast_convert_type(i, jnp.float32)
    for _ in range(3):
        y = y * (1.5 - 0.5 * x * y * y)
    return y


def _sc_edge(m_arr, table_a, table_b, senders, receivers, lns, lnb):
    @functools.partial(
        pl.kernel,
        out_type=jax.ShapeDtypeStruct((E, D), jnp.float32),
        mesh=_MESH,
        scratch_types=[
            pltpu.VMEM((CHUNK,), jnp.int32),
            pltpu.VMEM((CHUNK,), jnp.int32),
            pltpu.VMEM((CHUNK, D), jnp.float32),
            pltpu.VMEM((CHUNK, D), jnp.float32),
            pltpu.VMEM((CHUNK, D), jnp.float32),
            pltpu.VMEM((D,), jnp.float32),
            pltpu.VMEM((D,), jnp.float32),
            pltpu.SemaphoreType.DMA,
            pltpu.SemaphoreType.DMA,
            pltpu.SemaphoreType.DMA,
        ],
    )
    def k(m_hbm, ta_hbm, tb_hbm, s_hbm, r_hbm, lns_hbm, lnb_hbm, out_hbm,
          ia_v, ib_v, m_v, a_v, b_v, lns_v, lnb_v, sem_a, sem_b, sem_m):
        wid = lax.axis_index("s") * NC + lax.axis_index("c")
        base = wid * EPT

        pltpu.sync_copy(lns_hbm, lns_v)
        pltpu.sync_copy(lnb_hbm, lnb_v)
        sregs = [lns_v[pl.ds(16 * j, 16)] for j in range(8)]
        bregs = [lnb_v[pl.ds(16 * j, 16)] for j in range(8)]

        def body(jc, carry):
            off = base + jc * CHUNK
            pltpu.sync_copy(s_hbm.at[pl.ds(off, CHUNK)], ia_v)
            pltpu.sync_copy(r_hbm.at[pl.ds(off, CHUNK)], ib_v)
            cp_m = pltpu.async_copy(m_hbm.at[pl.ds(off, CHUNK)], m_v, sem_m)
            cp_a = pltpu.async_copy(ta_hbm.at[ia_v], a_v, sem_a)
            cp_b = pltpu.async_copy(tb_hbm.at[ib_v], b_v, sem_b)
            cp_m.wait()
            cp_a.wait()
            cp_b.wait()

            def row(r, carry2):
                xs = [m_v[r, pl.ds(16 * j, 16)] + a_v[r, pl.ds(16 * j, 16)]
                      + b_v[r, pl.ds(16 * j, 16)] for j in range(8)]
                sv = (((xs[0] + xs[1]) + (xs[2] + xs[3]))
                      + ((xs[4] + xs[5]) + (xs[6] + xs[7])))
                qs = [x * x for x in xs]
                qv = (((qs[0] + qs[1]) + (qs[2] + qs[3]))
                      + ((qs[4] + qs[5]) + (qs[6] + qs[7])))

                def lanesum(v):
                    p = [v[2 * t] + v[2 * t + 1] for t in range(8)]
                    p = [p[2 * t] + p[2 * t + 1] for t in range(4)]
                    p = [p[2 * t] + p[2 * t + 1] for t in range(2)]
                    return p[0] + p[1]

                mean = lanesum(sv) * (1.0 / D)
                var = lanesum(qv) * (1.0 / D) - mean * mean
                rsv = _rsqrt16(jnp.full((16,), var + 1e-6, jnp.float32))
                mv = jnp.full((16,), mean, jnp.float32)
                for j in range(8):
                    y = (xs[j] - mv) * (rsv * sregs[j]) + bregs[j]
                    m_v[r, pl.ds(16 * j, 16)] = jnp.maximum(y, 0.0)
                return carry2

            lax.fori_loop(0, CHUNK, row, 0)
            pltpu.sync_copy(m_v, out_hbm.at[pl.ds(off, CHUNK)])
            return carry

        lax.fori_loop(0, EPT // CHUNK, body, 0)

    return k(m_arr, table_a, table_b, senders, receivers, lns, lnb)


def _sc_segsum2(data, senders, receivers):
    @functools.partial(
        pl.kernel,
        out_type=[jax.ShapeDtypeStruct((NPAD, D), jnp.float32),
                  jax.ShapeDtypeStruct((NPAD, D), jnp.float32)],
        mesh=_MESH,
        scratch_types=[
            pltpu.VMEM((CHUNK,), jnp.int32),
            pltpu.VMEM((CHUNK, D), jnp.float32),
            pltpu.VMEM((CHUNK, D), jnp.float32),
            pltpu.VMEM_SHARED((NPAD, D), jnp.float32),
        ],
    )
    def k(d_hbm, s_hbm, r_hbm, sent_hbm, recv_hbm, idx_v, rows_v, zbuf, acc):
        core = lax.axis_index("c")
        sid = lax.axis_index("s")

        def zbody(kk, carry):
            i = kk // 8
            j = (kk % 8) * 16
            zbuf[i, pl.ds(j, 16)] = jnp.zeros((16,), jnp.float32)
            return carry

        lax.fori_loop(0, CHUNK * 8, zbody, 0)
        for t in range(STRIPE // CHUNK):
            pltpu.sync_copy(zbuf, acc.at[pl.ds(sid * STRIPE + t * CHUNK, CHUNK)])
        plsc.subcore_barrier()

        def make_body(idx_hbm):
            def body(j, carry):
                off = sid * EPS + j * CHUNK
                pltpu.sync_copy(idx_hbm.at[pl.ds(off, CHUNK)], idx_v)
                pltpu.sync_copy(d_hbm.at[pl.ds(off, CHUNK)], rows_v)
                pltpu.sync_copy(rows_v, acc.at[idx_v], add=True)
                return carry
            return body

        @pl.when(core == 0)
        def _():
            lax.fori_loop(0, EPS // CHUNK, make_body(s_hbm), 0)

        @pl.when(core == 1)
        def _():
            lax.fori_loop(0, EPS // CHUNK, make_body(r_hbm), 0)

        plsc.subcore_barrier()

        @pl.when(core == 0)
        def _():
            pltpu.sync_copy(acc.at[pl.ds(sid * STRIPE, STRIPE)],
                            sent_hbm.at[pl.ds(sid * STRIPE, STRIPE)])

        @pl.when(core == 1)
        def _():
            pltpu.sync_copy(acc.at[pl.ds(sid * STRIPE, STRIPE)],
                            recv_hbm.at[pl.ds(sid * STRIPE, STRIPE)])

    return k(data, senders, receivers)


def kernel(nodes, edge_attr, senders, receivers, train, params):
    del train
    senders = senders.astype(jnp.int32)
    receivers = receivers.astype(jnp.int32)

    h_n = _mm(nodes, params['en']['W'], params['en']['b'][None])
    h_e = _mm(edge_attr, params['ee']['W'], params['ee']['b'][None])
    g = jnp.zeros((1, D), jnp.float32)

    c_e = params['steps'][0]['e']['b'][None]
    c_n = params['steps'][0]['n']['b'][None]

    out = None
    for i, sp in enumerate(params['steps']):
        we = sp['e']['W']
        wn = sp['n']['W']

        a_tab, b_tab, t_arr = _node_pre(h_n, we[D:2 * D], we[2 * D:3 * D],
                                        wn[0:D])
        m_arr = _mm(h_e, we[0:D], c_e)
        h_e = _sc_edge(m_arr, a_tab, b_tab, senders, receivers,
                       sp['e']['ln_s'], sp['e']['ln_b'])
        sent, recv = _sc_segsum2(h_e, senders, receivers)
        h_n, nsum, esum = _node_update(
            t_arr, sent[:N], recv[:N], wn[D:2 * D], wn[2 * D:3 * D],
            c_n, sp['n']['ln_s'][None], sp['n']['ln_b'][None])

        gp = sp['g']
        if i + 1 < len(params['steps']):
            nxt = params['steps'][i + 1]
            g, c_e, c_n = _global_update(
                nsum, esum, g, gp['W'], gp['b'][None],
                gp['ln_s'][None], gp['ln_b'][None],
                nxt['e']['W'][3 * D:], nxt['e']['b'][None],
                nxt['n']['W'][3 * D:], nxt['n']['b'][None])
        else:
            out = _global_final(
                nsum, esum, g, gp['W'], gp['b'][None],
                gp['ln_s'][None], gp['ln_b'][None],
                params['dec']['W'], params['dec']['b'][None])

    return out


# speedup vs baseline: 1.2636x; 1.2636x over previous
import functools

import jax
import jax.numpy as jnp
from jax import lax
from jax.experimental import pallas as pl
from jax.experimental.pallas import tpu as pltpu
from jax.experimental.pallas import tpu_sc as plsc

N = 10000
E = 320000
D = 128

NC = 2
NS = 16
NW = NC * NS

NPAD = 10240
STRIPE = NPAD // NS

CHUNK = 80
EPT = E // NW
EPS = E // NS

_MESH = plsc.VectorSubcoreMesh(
    core_axis_name="c", subcore_axis_name="s", num_cores=NC, num_subcores=NS)


def _mm(x, w, c, br=2000):
    r, k = x.shape
    dout = w.shape[1]

    def body(x_ref, w_ref, c_ref, o_ref):
        o_ref[...] = (
            jnp.dot(x_ref[...], w_ref[...], preferred_element_type=jnp.float32)
            + c_ref[...])

    return pl.pallas_call(
        body,
        grid=(r // br,),
        in_specs=[
            pl.BlockSpec((br, k), lambda i: (i, 0)),
            pl.BlockSpec((k, dout), lambda i: (0, 0)),
            pl.BlockSpec((1, dout), lambda i: (0, 0)),
        ],
        out_specs=pl.BlockSpec((br, dout), lambda i: (i, 0)),
        out_shape=jax.ShapeDtypeStruct((r, dout), jnp.float32),
    )(x, w, c)


def _ln_relu(x, s, b):
    m = jnp.mean(x, axis=-1, keepdims=True)
    xc = x - m
    v = jnp.mean(xc * xc, axis=-1, keepdims=True)
    return jax.nn.relu(xc * lax.rsqrt(v + 1e-6) * s + b)


def _edge_finish(m_arr, ga, gb, s, b, br=2000):
    def body(m_ref, a_ref, b_ref, s_ref, bb_ref, o_ref):
        x = m_ref[...] + a_ref[...] + b_ref[...]
        o_ref[...] = _ln_relu(x, s_ref[...], bb_ref[...])

    return pl.pallas_call(
        body,
        grid=(E // br,),
        in_specs=[
            pl.BlockSpec((br, D), lambda i: (i, 0)),
            pl.BlockSpec((br, D), lambda i: (i, 0)),
            pl.BlockSpec((br, D), lambda i: (i, 0)),
            pl.BlockSpec((1, D), lambda i: (0, 0)),
            pl.BlockSpec((1, D), lambda i: (0, 0)),
        ],
        out_specs=pl.BlockSpec((br, D), lambda i: (i, 0)),
        out_shape=jax.ShapeDtypeStruct((E, D), jnp.float32),
    )(m_arr, ga, gb, s, b)


def _node_pre(h_n, w_a, w_b, w_t):
    def body(x_ref, wa_ref, wb_ref, wt_ref, a_ref, b_ref, t_ref):
        x = x_ref[...]
        a_ref[...] = jnp.dot(x, wa_ref[...], preferred_element_type=jnp.float32)
        b_ref[...] = jnp.dot(x, wb_ref[...], preferred_element_type=jnp.float32)
        t_ref[...] = jnp.dot(x, wt_ref[...], preferred_element_type=jnp.float32)

    br = 2000
    sds = jax.ShapeDtypeStruct((N, D), jnp.float32)
    return pl.pallas_call(
        body,
        grid=(N // br,),
        in_specs=[
            pl.BlockSpec((br, D), lambda i: (i, 0)),
            pl.BlockSpec((D, D), lambda i: (0, 0)),
            pl.BlockSpec((D, D), lambda i: (0, 0)),
            pl.BlockSpec((D, D), lambda i: (0, 0)),
        ],
        out_specs=[
            pl.BlockSpec((br, D), lambda i: (i, 0)),
            pl.BlockSpec((br, D), lambda i: (i, 0)),
            pl.BlockSpec((br, D), lambda i: (i, 0)),
        ],
        out_shape=[sds, sds, sds],
    )(h_n, w_a, w_b, w_t)


def _node_update(t, sent, recv, w_s, w_r, c, s, b):
    br = 2000

    def body(t_ref, sp_ref, rp_ref, ws_ref, wr_ref, c_ref, s_ref, b_ref,
             o_ref, nsum_ref, esum_ref):
        i = pl.program_id(0)
        sent_blk = sp_ref[...]
        recv_blk = rp_ref[...]
        x = (t_ref[...]
             + jnp.dot(sent_blk, ws_ref[...], preferred_element_type=jnp.float32)
             + jnp.dot(recv_blk, wr_ref[...], preferred_element_type=jnp.float32)
             + c_ref[...])
        h = _ln_relu(x, s_ref[...], b_ref[...])
        o_ref[...] = h

        @pl.when(i == 0)
        def _():
            nsum_ref[...] = jnp.zeros_like(nsum_ref)
            esum_ref[...] = jnp.zeros_like(esum_ref)

        nsum_ref[...] += jnp.sum(h, axis=0, keepdims=True)
        esum_ref[...] += jnp.sum(sent_blk, axis=0, keepdims=True)

    one = jax.ShapeDtypeStruct((1, D), jnp.float32)
    return pl.pallas_call(
        body,
        grid=(N // br,),
        in_specs=[
            pl.BlockSpec((br, D), lambda i: (i, 0)),
            pl.BlockSpec((br, D), lambda i: (i, 0)),
            pl.BlockSpec((br, D), lambda i: (i, 0)),
            pl.BlockSpec((D, D), lambda i: (0, 0)),
            pl.BlockSpec((D, D), lambda i: (0, 0)),
            pl.BlockSpec((1, D), lambda i: (0, 0)),
            pl.BlockSpec((1, D), lambda i: (0, 0)),
            pl.BlockSpec((1, D), lambda i: (0, 0)),
        ],
        out_specs=[
            pl.BlockSpec((br, D), lambda i: (i, 0)),
            pl.BlockSpec((1, D), lambda i: (0, 0)),
            pl.BlockSpec((1, D), lambda i: (0, 0)),
        ],
        out_shape=[jax.ShapeDtypeStruct((N, D), jnp.float32), one, one],
    )(t, sent, recv, w_s, w_r, c, s, b)


def _global_update(nsum, esum, g, wg, bg, lns, lnb, w_e_g, be, w_n_g, bn):
    def body(ns_ref, es_ref, g_ref, wg_ref, bg_ref, s_ref, b_ref,
             weg_ref, be_ref, wng_ref, bn_ref, g_out, ce_out, cn_out):
        wg = wg_ref[...]
        x = (jnp.dot(ns_ref[...], wg[0:D, :], preferred_element_type=jnp.float32)
             + jnp.dot(es_ref[...], wg[D:2 * D, :], preferred_element_type=jnp.float32)
             + jnp.dot(g_ref[...], wg[2 * D:3 * D, :], preferred_element_type=jnp.float32)
             + bg_ref[...])
        gn = _ln_relu(x, s_ref[...], b_ref[...])
        g_out[...] = gn
        ce_out[...] = jnp.dot(gn, weg_ref[...], preferred_element_type=jnp.float32) + be_ref[...]
        cn_out[...] = jnp.dot(gn, wng_ref[...], preferred_element_type=jnp.float32) + bn_ref[...]

    one = jax.ShapeDtypeStruct((1, D), jnp.float32)
    return pl.pallas_call(
        body,
        out_shape=[one, one, one],
    )(nsum, esum, g, wg, bg, lns, lnb, w_e_g, be, w_n_g, bn)


def _global_final(nsum, esum, g, wg, bg, lns, lnb, wdec, bdec):
    def body(ns_ref, es_ref, g_ref, wg_ref, bg_ref, s_ref, b_ref,
             wd_ref, bd_ref, o_ref):
        wg = wg_ref[...]
        x = (jnp.dot(ns_ref[...], wg[0:D, :], preferred_element_type=jnp.float32)
             + jnp.dot(es_ref[...], wg[D:2 * D, :], preferred_element_type=jnp.float32)
             + jnp.dot(g_ref[...], wg[2 * D:3 * D, :], preferred_element_type=jnp.float32)
             + bg_ref[...])
        gn = _ln_relu(x, s_ref[...], b_ref[...])
        o_ref[...] = jnp.dot(gn, wd_ref[...], preferred_element_type=jnp.float32) + bd_ref[...]

    return pl.pallas_call(
        body,
        out_shape=jax.ShapeDtypeStruct((1, D), jnp.float32),
    )(nsum, esum, g, wg, bg, lns, lnb, wdec, bdec)


def _rsqrt16(x):
    i = lax.bitcast_convert_type(x, jnp.int32)
    i = jnp.int32(0x5F3759DF) - lax.shift_right_arithmetic(i, jnp.int32(1))
    y = lax.bitcast_convert_type(i, jnp.float32)
    for _ in range(3):
        y = y * (1.5 - 0.5 * x * y * y)
    return y


def _sc_edge(m_arr, table_a, table_b, senders, receivers, lns, lnb):
    @functools.partial(
        pl.kernel,
        out_type=jax.ShapeDtypeStruct((E, D), jnp.float32),
        mesh=_MESH,
        scratch_types=[
            pltpu.VMEM((CHUNK,), jnp.int32),
            pltpu.VMEM((CHUNK,), jnp.int32),
            pltpu.VMEM((CHUNK, D), jnp.float32),
            pltpu.VMEM((CHUNK, D), jnp.float32),
            pltpu.VMEM((CHUNK, D), jnp.float32),
            pltpu.VMEM((D,), jnp.float32),
            pltpu.VMEM((D,), jnp.float32),
            pltpu.SemaphoreType.DMA,
            pltpu.SemaphoreType.DMA,
            pltpu.SemaphoreType.DMA,
        ],
    )
    def k(m_hbm, ta_hbm, tb_hbm, s_hbm, r_hbm, lns_hbm, lnb_hbm, out_hbm,
          ia_v, ib_v, m_v, a_v, b_v, lns_v, lnb_v, sem_a, sem_b, sem_m):
        wid = lax.axis_index("s") * NC + lax.axis_index("c")
        base = wid * EPT

        pltpu.sync_copy(lns_hbm, lns_v)
        pltpu.sync_copy(lnb_hbm, lnb_v)
        sregs = [lns_v[pl.ds(16 * j, 16)] for j in range(8)]
        bregs = [lnb_v[pl.ds(16 * j, 16)] for j in range(8)]

        def body(jc, carry):
            off = base + jc * CHUNK
            pltpu.sync_copy(s_hbm.at[pl.ds(off, CHUNK)], ia_v)
            pltpu.sync_copy(r_hbm.at[pl.ds(off, CHUNK)], ib_v)
            cp_m = pltpu.async_copy(m_hbm.at[pl.ds(off, CHUNK)], m_v, sem_m)
            cp_a = pltpu.async_copy(ta_hbm.at[ia_v], a_v, sem_a)
            cp_b = pltpu.async_copy(tb_hbm.at[ib_v], b_v, sem_b)
            cp_m.wait()
            cp_a.wait()
            cp_b.wait()

            @plsc.parallel_loop(0, CHUNK, unroll=4)
            def row(r):
                xs = [m_v[r, pl.ds(16 * j, 16)] + a_v[r, pl.ds(16 * j, 16)]
                      + b_v[r, pl.ds(16 * j, 16)] for j in range(8)]
                sv = (((xs[0] + xs[1]) + (xs[2] + xs[3]))
                      + ((xs[4] + xs[5]) + (xs[6] + xs[7])))
                qs = [x * x for x in xs]
                qv = (((qs[0] + qs[1]) + (qs[2] + qs[3]))
                      + ((qs[4] + qs[5]) + (qs[6] + qs[7])))

                def lanesum(v):
                    p = [v[2 * t] + v[2 * t + 1] for t in range(8)]
                    p = [p[2 * t] + p[2 * t + 1] for t in range(4)]
                    p = [p[2 * t] + p[2 * t + 1] for t in range(2)]
                    return p[0] + p[1]

                mean = lanesum(sv) * (1.0 / D)
                var = lanesum(qv) * (1.0 / D) - mean * mean
                rsv = _rsqrt16(jnp.full((16,), var + 1e-6, jnp.float32))
                mv = jnp.full((16,), mean, jnp.float32)
                for j in range(8):
                    y = (xs[j] - mv) * (rsv * sregs[j]) + bregs[j]
                    m_v[r, pl.ds(16 * j, 16)] = jnp.maximum(y, 0.0)

            pltpu.sync_copy(m_v, out_hbm.at[pl.ds(off, CHUNK)])
            return carry

        lax.fori_loop(0, EPT // CHUNK, body, 0)

    return k(m_arr, table_a, table_b, senders, receivers, lns, lnb)


def _sc_segsum2(data, senders, receivers):
    @functools.partial(
        pl.kernel,
        out_type=[jax.ShapeDtypeStruct((NPAD, D), jnp.float32),
                  jax.ShapeDtypeStruct((NPAD, D), jnp.float32)],
        mesh=_MESH,
        scratch_types=[
            pltpu.VMEM((CHUNK,), jnp.int32),
            pltpu.VMEM((CHUNK, D), jnp.float32),
            pltpu.VMEM((CHUNK, D), jnp.float32),
            pltpu.VMEM_SHARED((NPAD, D), jnp.float32),
        ],
    )
    def k(d_hbm, s_hbm, r_hbm, sent_hbm, recv_hbm, idx_v, rows_v, zbuf, acc):
        core = lax.axis_index("c")
        sid = lax.axis_index("s")

        def zbody(kk, carry):
            i = kk // 8
            j = (kk % 8) * 16
            zbuf[i, pl.ds(j, 16)] = jnp.zeros((16,), jnp.float32)
            return carry

        lax.fori_loop(0, CHUNK * 8, zbody, 0)
        for t in range(STRIPE // CHUNK):
            pltpu.sync_copy(zbuf, acc.at[pl.ds(sid * STRIPE + t * CHUNK, CHUNK)])
        plsc.subcore_barrier()

        def make_body(idx_hbm):
            def body(j, carry):
                off = sid * EPS + j * CHUNK
                pltpu.sync_copy(idx_hbm.at[pl.ds(off, CHUNK)], idx_v)
                pltpu.sync_copy(d_hbm.at[pl.ds(off, CHUNK)], rows_v)
                pltpu.sync_copy(rows_v, acc.at[idx_v], add=True)
                return carry
            return body

        @pl.when(core == 0)
        def _():
            lax.fori_loop(0, EPS // CHUNK, make_body(s_hbm), 0)

        @pl.when(core == 1)
        def _():
            lax.fori_loop(0, EPS // CHUNK, make_body(r_hbm), 0)

        plsc.subcore_barrier()

        @pl.when(core == 0)
        def _():
            pltpu.sync_copy(acc.at[pl.ds(sid * STRIPE, STRIPE)],
                            sent_hbm.at[pl.ds(sid * STRIPE, STRIPE)])

        @pl.when(core == 1)
        def _():
            pltpu.sync_copy(acc.at[pl.ds(sid * STRIPE, STRIPE)],
                            recv_hbm.at[pl.ds(sid * STRIPE, STRIPE)])

    return k(data, senders, receivers)


def kernel(nodes, edge_attr, senders, receivers, train, params):
    del train
    senders = senders.astype(jnp.int32)
    receivers = receivers.astype(jnp.int32)

    h_n = _mm(nodes, params['en']['W'], params['en']['b'][None])
    h_e = _mm(edge_attr, params['ee']['W'], params['ee']['b'][None])
    g = jnp.zeros((1, D), jnp.float32)

    c_e = params['steps'][0]['e']['b'][None]
    c_n = params['steps'][0]['n']['b'][None]

    out = None
    for i, sp in enumerate(params['steps']):
        we = sp['e']['W']
        wn = sp['n']['W']

        a_tab, b_tab, t_arr = _node_pre(h_n, we[D:2 * D], we[2 * D:3 * D],
                                        wn[0:D])
        m_arr = _mm(h_e, we[0:D], c_e)
        h_e = _sc_edge(m_arr, a_tab, b_tab, senders, receivers,
                       sp['e']['ln_s'], sp['e']['ln_b'])
        sent, recv = _sc_segsum2(h_e, senders, receivers)
        h_n, nsum, esum = _node_update(
            t_arr, sent[:N], recv[:N], wn[D:2 * D], wn[2 * D:3 * D],
            c_n, sp['n']['ln_s'][None], sp['n']['ln_b'][None])

        gp = sp['g']
        if i + 1 < len(params['steps']):
            nxt = params['steps'][i + 1]
            g, c_e, c_n = _global_update(
                nsum, esum, g, gp['W'], gp['b'][None],
                gp['ln_s'][None], gp['ln_b'][None],
                nxt['e']['W'][3 * D:], nxt['e']['b'][None],
                nxt['n']['W'][3 * D:], nxt['n']['b'][None])
        else:
            out = _global_final(
                nsum, esum, g, gp['W'], gp['b'][None],
                gp['ln_s'][None], gp['ln_b'][None],
                params['dec']['W'], params['dec']['b'][None])

    return out


# speedup vs baseline: 1.6781x; 1.3280x over previous
import functools

import jax
import jax.numpy as jnp
from jax import lax
from jax.experimental import pallas as pl
from jax.experimental.pallas import tpu as pltpu
from jax.experimental.pallas import tpu_sc as plsc

N = 10000
E = 320000
D = 128

NC = 2
NS = 16
NW = NC * NS

NPAD = 10240
STRIPE = NPAD // NS

CHUNK = 80
EPT = E // NW
EPS = E // NS

_MESH = plsc.VectorSubcoreMesh(
    core_axis_name="c", subcore_axis_name="s", num_cores=NC, num_subcores=NS)


def _mm(x, w, c, br=2000):
    r, k = x.shape
    dout = w.shape[1]

    def body(x_ref, w_ref, c_ref, o_ref):
        o_ref[...] = (
            jnp.dot(x_ref[...], w_ref[...], preferred_element_type=jnp.float32)
            + c_ref[...])

    return pl.pallas_call(
        body,
        grid=(r // br,),
        in_specs=[
            pl.BlockSpec((br, k), lambda i: (i, 0)),
            pl.BlockSpec((k, dout), lambda i: (0, 0)),
            pl.BlockSpec((1, dout), lambda i: (0, 0)),
        ],
        out_specs=pl.BlockSpec((br, dout), lambda i: (i, 0)),
        out_shape=jax.ShapeDtypeStruct((r, dout), jnp.float32),
    )(x, w, c)


def _ln_relu(x, s, b):
    m = jnp.mean(x, axis=-1, keepdims=True)
    xc = x - m
    v = jnp.mean(xc * xc, axis=-1, keepdims=True)
    return jax.nn.relu(xc * lax.rsqrt(v + 1e-6) * s + b)


def _edge_finish(m_arr, ga, gb, s, b, br=2000):
    def body(m_ref, a_ref, b_ref, s_ref, bb_ref, o_ref):
        x = m_ref[...] + a_ref[...] + b_ref[...]
        o_ref[...] = _ln_relu(x, s_ref[...], bb_ref[...])

    return pl.pallas_call(
        body,
        grid=(E // br,),
        in_specs=[
            pl.BlockSpec((br, D), lambda i: (i, 0)),
            pl.BlockSpec((br, D), lambda i: (i, 0)),
            pl.BlockSpec((br, D), lambda i: (i, 0)),
            pl.BlockSpec((1, D), lambda i: (0, 0)),
            pl.BlockSpec((1, D), lambda i: (0, 0)),
        ],
        out_specs=pl.BlockSpec((br, D), lambda i: (i, 0)),
        out_shape=jax.ShapeDtypeStruct((E, D), jnp.float32),
    )(m_arr, ga, gb, s, b)


def _node_pre(h_n, w_a, w_b, w_t):
    def body(x_ref, wa_ref, wb_ref, wt_ref, a_ref, b_ref, t_ref):
        x = x_ref[...]
        a_ref[...] = jnp.dot(x, wa_ref[...], preferred_element_type=jnp.float32)
        b_ref[...] = jnp.dot(x, wb_ref[...], preferred_element_type=jnp.float32)
        t_ref[...] = jnp.dot(x, wt_ref[...], preferred_element_type=jnp.float32)

    br = 2000
    sds = jax.ShapeDtypeStruct((N, D), jnp.float32)
    return pl.pallas_call(
        body,
        grid=(N // br,),
        in_specs=[
            pl.BlockSpec((br, D), lambda i: (i, 0)),
            pl.BlockSpec((D, D), lambda i: (0, 0)),
            pl.BlockSpec((D, D), lambda i: (0, 0)),
            pl.BlockSpec((D, D), lambda i: (0, 0)),
        ],
        out_specs=[
            pl.BlockSpec((br, D), lambda i: (i, 0)),
            pl.BlockSpec((br, D), lambda i: (i, 0)),
            pl.BlockSpec((br, D), lambda i: (i, 0)),
        ],
        out_shape=[sds, sds, sds],
    )(h_n, w_a, w_b, w_t)


def _node_update(t, sent, recv, w_s, w_r, c, s, b):
    br = 2000

    def body(t_ref, sp_ref, rp_ref, ws_ref, wr_ref, c_ref, s_ref, b_ref,
             o_ref, nsum_ref, esum_ref):
        i = pl.program_id(0)
        sent_blk = sp_ref[...]
        recv_blk = rp_ref[...]
        x = (t_ref[...]
             + jnp.dot(sent_blk, ws_ref[...], preferred_element_type=jnp.float32)
             + jnp.dot(recv_blk, wr_ref[...], preferred_element_type=jnp.float32)
             + c_ref[...])
        h = _ln_relu(x, s_ref[...], b_ref[...])
        o_ref[...] = h

        @pl.when(i == 0)
        def _():
            nsum_ref[...] = jnp.zeros_like(nsum_ref)
            esum_ref[...] = jnp.zeros_like(esum_ref)

        nsum_ref[...] += jnp.sum(h, axis=0, keepdims=True)
        esum_ref[...] += jnp.sum(sent_blk, axis=0, keepdims=True)

    one = jax.ShapeDtypeStruct((1, D), jnp.float32)
    return pl.pallas_call(
        body,
        grid=(N // br,),
        in_specs=[
            pl.BlockSpec((br, D), lambda i: (i, 0)),
            pl.BlockSpec((br, D), lambda i: (i, 0)),
            pl.BlockSpec((br, D), lambda i: (i, 0)),
            pl.BlockSpec((D, D), lambda i: (0, 0)),
            pl.BlockSpec((D, D), lambda i: (0, 0)),
            pl.BlockSpec((1, D), lambda i: (0, 0)),
            pl.BlockSpec((1, D), lambda i: (0, 0)),
            pl.BlockSpec((1, D), lambda i: (0, 0)),
        ],
        out_specs=[
            pl.BlockSpec((br, D), lambda i: (i, 0)),
            pl.BlockSpec((1, D), lambda i: (0, 0)),
            pl.BlockSpec((1, D), lambda i: (0, 0)),
        ],
        out_shape=[jax.ShapeDtypeStruct((N, D), jnp.float32), one, one],
    )(t, sent, recv, w_s, w_r, c, s, b)


def _global_update(nsum, esum, g, wg, bg, lns, lnb, w_e_g, be, w_n_g, bn):
    def body(ns_ref, es_ref, g_ref, wg_ref, bg_ref, s_ref, b_ref,
             weg_ref, be_ref, wng_ref, bn_ref, g_out, ce_out, cn_out):
        wg = wg_ref[...]
        x = (jnp.dot(ns_ref[...], wg[0:D, :], preferred_element_type=jnp.float32)
             + jnp.dot(es_ref[...], wg[D:2 * D, :], preferred_element_type=jnp.float32)
             + jnp.dot(g_ref[...], wg[2 * D:3 * D, :], preferred_element_type=jnp.float32)
             + bg_ref[...])
        gn = _ln_relu(x, s_ref[...], b_ref[...])
        g_out[...] = gn
        ce_out[...] = jnp.dot(gn, weg_ref[...], preferred_element_type=jnp.float32) + be_ref[...]
        cn_out[...] = jnp.dot(gn, wng_ref[...], preferred_element_type=jnp.float32) + bn_ref[...]

    one = jax.ShapeDtypeStruct((1, D), jnp.float32)
    return pl.pallas_call(
        body,
        out_shape=[one, one, one],
    )(nsum, esum, g, wg, bg, lns, lnb, w_e_g, be, w_n_g, bn)


def _global_final(nsum, esum, g, wg, bg, lns, lnb, wdec, bdec):
    def body(ns_ref, es_ref, g_ref, wg_ref, bg_ref, s_ref, b_ref,
             wd_ref, bd_ref, o_ref):
        wg = wg_ref[...]
        x = (jnp.dot(ns_ref[...], wg[0:D, :], preferred_element_type=jnp.float32)
             + jnp.dot(es_ref[...], wg[D:2 * D, :], preferred_element_type=jnp.float32)
             + jnp.dot(g_ref[...], wg[2 * D:3 * D, :], preferred_element_type=jnp.float32)
             + bg_ref[...])
        gn = _ln_relu(x, s_ref[...], b_ref[...])
        o_ref[...] = jnp.dot(gn, wd_ref[...], preferred_element_type=jnp.float32) + bd_ref[...]

    return pl.pallas_call(
        body,
        out_shape=jax.ShapeDtypeStruct((1, D), jnp.float32),
    )(nsum, esum, g, wg, bg, lns, lnb, wdec, bdec)


def _rsqrt16(x):
    i = lax.bitcast_convert_type(x, jnp.int32)
    i = jnp.int32(0x5F3759DF) - lax.shift_right_arithmetic(i, jnp.int32(1))
    y = lax.bitcast_convert_type(i, jnp.float32)
    for _ in range(3):
        y = y * (1.5 - 0.5 * x * y * y)
    return y


def _sc_edge(m_arr, table_a, table_b, senders, receivers, lns, lnb):
    nchunks = EPT // CHUNK
    buf2 = lambda shape, dt: [pltpu.VMEM(shape, dt), pltpu.VMEM(shape, dt)]

    @functools.partial(
        pl.kernel,
        out_type=jax.ShapeDtypeStruct((E, D), jnp.float32),
        mesh=_MESH,
        scratch_types=(
            buf2((CHUNK,), jnp.int32) + buf2((CHUNK,), jnp.int32)
            + buf2((CHUNK, D), jnp.float32) + buf2((CHUNK, D), jnp.float32)
            + buf2((CHUNK, D), jnp.float32) + buf2((CHUNK, D), jnp.float32)
            + [pltpu.VMEM((D,), jnp.float32), pltpu.VMEM((D,), jnp.float32)]
            + [pltpu.SemaphoreType.DMA] * 12
        ),
    )
    def k(m_hbm, ta_hbm, tb_hbm, s_hbm, r_hbm, lns_hbm, lnb_hbm, out_hbm,
          ia0, ia1, ib0, ib1, m0, m1, a0, a1, b0, b1, o0, o1,
          lns_v, lnb_v,
          sia0, sia1, sib0, sib1, sm0, sm1, sa0, sa1, sb0, sb1, so0, so1):
        wid = lax.axis_index("s") * NC + lax.axis_index("c")
        base = wid * EPT
        B = [dict(ia=ia0, ib=ib0, m=m0, a=a0, b=b0, o=o0, sia=sia0, sib=sib0,
                  sm=sm0, sa=sa0, sb=sb0, so=so0),
             dict(ia=ia1, ib=ib1, m=m1, a=a1, b=b1, o=o1, sia=sia1, sib=sib1,
                  sm=sm1, sa=sa1, sb=sb1, so=so1)]

        pltpu.sync_copy(lns_hbm, lns_v)
        pltpu.sync_copy(lnb_hbm, lnb_v)
        sregs = [lns_v[pl.ds(16 * j, 16)] for j in range(8)]
        bregs = [lnb_v[pl.ds(16 * j, 16)] for j in range(8)]

        def offs(c):
            return base + c * CHUNK

        def issue_idx(c, P):
            pltpu.async_copy(s_hbm.at[pl.ds(offs(c), CHUNK)], P['ia'], P['sia'])
            pltpu.async_copy(r_hbm.at[pl.ds(offs(c), CHUNK)], P['ib'], P['sib'])

        def wait_idx(c, P):
            pltpu.make_async_copy(s_hbm.at[pl.ds(offs(c), CHUNK)], P['ia'], P['sia']).wait()
            pltpu.make_async_copy(r_hbm.at[pl.ds(offs(c), CHUNK)], P['ib'], P['sib']).wait()

        def issue_main(c, P):
            pltpu.async_copy(m_hbm.at[pl.ds(offs(c), CHUNK)], P['m'], P['sm'])
            pltpu.async_copy(ta_hbm.at[P['ia']], P['a'], P['sa'])
            pltpu.async_copy(tb_hbm.at[P['ib']], P['b'], P['sb'])

        def wait_main(c, P):
            pltpu.make_async_copy(m_hbm.at[pl.ds(offs(c), CHUNK)], P['m'], P['sm']).wait()
            pltpu.make_async_copy(ta_hbm.at[P['ia']], P['a'], P['sa']).wait()
            pltpu.make_async_copy(tb_hbm.at[P['ib']], P['b'], P['sb']).wait()

        def issue_out(c, P):
            pltpu.async_copy(P['o'], out_hbm.at[pl.ds(offs(c), CHUNK)], P['so'])

        def wait_out(c, P):
            pltpu.make_async_copy(P['o'], out_hbm.at[pl.ds(offs(c), CHUNK)], P['so']).wait()

        def compute(P):
            m_v, a_v, b_v, o_v = P['m'], P['a'], P['b'], P['o']

            @plsc.parallel_loop(0, CHUNK, unroll=4)
            def row(r):
                xs = [m_v[r, pl.ds(16 * j, 16)] + a_v[r, pl.ds(16 * j, 16)]
                      + b_v[r, pl.ds(16 * j, 16)] for j in range(8)]
                sv = (((xs[0] + xs[1]) + (xs[2] + xs[3]))
                      + ((xs[4] + xs[5]) + (xs[6] + xs[7])))
                qs = [x * x for x in xs]
                qv = (((qs[0] + qs[1]) + (qs[2] + qs[3]))
                      + ((qs[4] + qs[5]) + (qs[6] + qs[7])))

                def lanesum(v):
                    p = [v[2 * t] + v[2 * t + 1] for t in range(8)]
                    p = [p[2 * t] + p[2 * t + 1] for t in range(4)]
                    p = [p[2 * t] + p[2 * t + 1] for t in range(2)]
                    return p[0] + p[1]

                mean = lanesum(sv) * (1.0 / D)
                var = lanesum(qv) * (1.0 / D) - mean * mean
                rsv = _rsqrt16(jnp.full((16,), var + 1e-6, jnp.float32))
                mv = jnp.full((16,), mean, jnp.float32)
                for j in range(8):
                    y = (xs[j] - mv) * (rsv * sregs[j]) + bregs[j]
                    o_v[r, pl.ds(16 * j, 16)] = jnp.maximum(y, 0.0)

        issue_idx(0, B[0])
        wait_idx(0, B[0])
        issue_main(0, B[0])
        issue_idx(1, B[1])

        def body(kk, carry):
            for bsel in (0, 1):
                c = 2 * kk + bsel
                P, Q = B[bsel], B[1 - bsel]
                wait_idx(c + 1, Q)
                issue_main(c + 1, Q)
                wait_main(c, P)

                @pl.when(c + 2 < nchunks)
                def _():
                    issue_idx(c + 2, P)

                @pl.when(c >= 2)
                def _():
                    wait_out(c - 2, P)

                compute(P)
                issue_out(c, P)
            return carry

        lax.fori_loop(0, (nchunks - 1) // 2, body, 0)

        c_last = nchunks - 1
        wait_main(c_last, B[0])
        wait_out(c_last - 2, B[0])
        compute(B[0])
        issue_out(c_last, B[0])
        wait_out(c_last - 1, B[1])
        wait_out(c_last, B[0])

    return k(m_arr, table_a, table_b, senders, receivers, lns, lnb)


def _sc_segsum2(data, senders, receivers):
    @functools.partial(
        pl.kernel,
        out_type=[jax.ShapeDtypeStruct((NPAD, D), jnp.float32),
                  jax.ShapeDtypeStruct((NPAD, D), jnp.float32)],
        mesh=_MESH,
        scratch_types=[
            pltpu.VMEM((CHUNK,), jnp.int32),
            pltpu.VMEM((CHUNK, D), jnp.float32),
            pltpu.VMEM((CHUNK, D), jnp.float32),
            pltpu.VMEM_SHARED((NPAD, D), jnp.float32),
        ],
    )
    def k(d_hbm, s_hbm, r_hbm, sent_hbm, recv_hbm, idx_v, rows_v, zbuf, acc):
        core = lax.axis_index("c")
        sid = lax.axis_index("s")

        def zbody(kk, carry):
            i = kk // 8
            j = (kk % 8) * 16
            zbuf[i, pl.ds(j, 16)] = jnp.zeros((16,), jnp.float32)
            return carry

        lax.fori_loop(0, CHUNK * 8, zbody, 0)
        for t in range(STRIPE // CHUNK):
            pltpu.sync_copy(zbuf, acc.at[pl.ds(sid * STRIPE + t * CHUNK, CHUNK)])
        plsc.subcore_barrier()

        def make_body(idx_hbm):
            def body(j, carry):
                off = sid * EPS + j * CHUNK
                pltpu.sync_copy(idx_hbm.at[pl.ds(off, CHUNK)], idx_v)
                pltpu.sync_copy(d_hbm.at[pl.ds(off, CHUNK)], rows_v)
                pltpu.sync_copy(rows_v, acc.at[idx_v], add=True)
                return carry
            return body

        @pl.when(core == 0)
        def _():
            lax.fori_loop(0, EPS // CHUNK, make_body(s_hbm), 0)

        @pl.when(core == 1)
        def _():
            lax.fori_loop(0, EPS // CHUNK, make_body(r_hbm), 0)

        plsc.subcore_barrier()

        @pl.when(core == 0)
        def _():
            pltpu.sync_copy(acc.at[pl.ds(sid * STRIPE, STRIPE)],
                            sent_hbm.at[pl.ds(sid * STRIPE, STRIPE)])

        @pl.when(core == 1)
        def _():
            pltpu.sync_copy(acc.at[pl.ds(sid * STRIPE, STRIPE)],
                            recv_hbm.at[pl.ds(sid * STRIPE, STRIPE)])

    return k(data, senders, receivers)


def kernel(nodes, edge_attr, senders, receivers, train, params):
    del train
    senders = senders.astype(jnp.int32)
    receivers = receivers.astype(jnp.int32)

    h_n = _mm(nodes, params['en']['W'], params['en']['b'][None])
    h_e = _mm(edge_attr, params['ee']['W'], params['ee']['b'][None])
    g = jnp.zeros((1, D), jnp.float32)

    c_e = params['steps'][0]['e']['b'][None]
    c_n = params['steps'][0]['n']['b'][None]

    out = None
    for i, sp in enumerate(params['steps']):
        we = sp['e']['W']
        wn = sp['n']['W']

        a_tab, b_tab, t_arr = _node_pre(h_n, we[D:2 * D], we[2 * D:3 * D],
                                        wn[0:D])
        m_arr = _mm(h_e, we[0:D], c_e)
        h_e = _sc_edge(m_arr, a_tab, b_tab, senders, receivers,
                       sp['e']['ln_s'], sp['e']['ln_b'])
        sent, recv = _sc_segsum2(h_e, senders, receivers)
        h_n, nsum, esum = _node_update(
            t_arr, sent[:N], recv[:N], wn[D:2 * D], wn[2 * D:3 * D],
            c_n, sp['n']['ln_s'][None], sp['n']['ln_b'][None])

        gp = sp['g']
        if i + 1 < len(params['steps']):
            nxt = params['steps'][i + 1]
            g, c_e, c_n = _global_update(
                nsum, esum, g, gp['W'], gp['b'][None],
                gp['ln_s'][None], gp['ln_b'][None],
                nxt['e']['W'][3 * D:], nxt['e']['b'][None],
                nxt['n']['W'][3 * D:], nxt['n']['b'][None])
        else:
            out = _global_final(
                nsum, esum, g, gp['W'], gp['b'][None],
                gp['ln_s'][None], gp['ln_b'][None],
                params['dec']['W'], params['dec']['b'][None])

    return out


# speedup vs baseline: 2.0416x; 1.2166x over previous
import functools

import jax
import jax.numpy as jnp
from jax import lax
from jax.experimental import pallas as pl
from jax.experimental.pallas import tpu as pltpu
from jax.experimental.pallas import tpu_sc as plsc

N = 10000
E = 320000
D = 128

NC = 2
NS = 16
NW = NC * NS

NPAD = 10240
STRIPE = NPAD // NS

CHUNK = 80
EPT = E // NW
EPS = E // NS

_MESH = plsc.VectorSubcoreMesh(
    core_axis_name="c", subcore_axis_name="s", num_cores=NC, num_subcores=NS)


def _mm(x, w, c, br=2000):
    r, k = x.shape
    dout = w.shape[1]

    def body(x_ref, w_ref, c_ref, o_ref):
        o_ref[...] = (
            jnp.dot(x_ref[...], w_ref[...], preferred_element_type=jnp.float32)
            + c_ref[...])

    return pl.pallas_call(
        body,
        grid=(r // br,),
        in_specs=[
            pl.BlockSpec((br, k), lambda i: (i, 0)),
            pl.BlockSpec((k, dout), lambda i: (0, 0)),
            pl.BlockSpec((1, dout), lambda i: (0, 0)),
        ],
        out_specs=pl.BlockSpec((br, dout), lambda i: (i, 0)),
        out_shape=jax.ShapeDtypeStruct((r, dout), jnp.float32),
    )(x, w, c)


def _ln_relu(x, s, b):
    m = jnp.mean(x, axis=-1, keepdims=True)
    xc = x - m
    v = jnp.mean(xc * xc, axis=-1, keepdims=True)
    return jax.nn.relu(xc * lax.rsqrt(v + 1e-6) * s + b)


def _edge_finish(m_arr, ga, gb, s, b, br=2000):
    def body(m_ref, a_ref, b_ref, s_ref, bb_ref, o_ref):
        x = m_ref[...] + a_ref[...] + b_ref[...]
        o_ref[...] = _ln_relu(x, s_ref[...], bb_ref[...])

    return pl.pallas_call(
        body,
        grid=(E // br,),
        in_specs=[
            pl.BlockSpec((br, D), lambda i: (i, 0)),
            pl.BlockSpec((br, D), lambda i: (i, 0)),
            pl.BlockSpec((br, D), lambda i: (i, 0)),
            pl.BlockSpec((1, D), lambda i: (0, 0)),
            pl.BlockSpec((1, D), lambda i: (0, 0)),
        ],
        out_specs=pl.BlockSpec((br, D), lambda i: (i, 0)),
        out_shape=jax.ShapeDtypeStruct((E, D), jnp.float32),
    )(m_arr, ga, gb, s, b)


def _node_pre(h_n, w_a, w_b, w_t):
    def body(x_ref, wa_ref, wb_ref, wt_ref, a_ref, b_ref, t_ref):
        x = x_ref[...]
        a_ref[...] = jnp.dot(x, wa_ref[...], preferred_element_type=jnp.float32)
        b_ref[...] = jnp.dot(x, wb_ref[...], preferred_element_type=jnp.float32)
        t_ref[...] = jnp.dot(x, wt_ref[...], preferred_element_type=jnp.float32)

    br = 2000
    sds = jax.ShapeDtypeStruct((N, D), jnp.float32)
    return pl.pallas_call(
        body,
        grid=(N // br,),
        in_specs=[
            pl.BlockSpec((br, D), lambda i: (i, 0)),
            pl.BlockSpec((D, D), lambda i: (0, 0)),
            pl.BlockSpec((D, D), lambda i: (0, 0)),
            pl.BlockSpec((D, D), lambda i: (0, 0)),
        ],
        out_specs=[
            pl.BlockSpec((br, D), lambda i: (i, 0)),
            pl.BlockSpec((br, D), lambda i: (i, 0)),
            pl.BlockSpec((br, D), lambda i: (i, 0)),
        ],
        out_shape=[sds, sds, sds],
    )(h_n, w_a, w_b, w_t)


def _node_update(t, sent, recv, w_s, w_r, c, s, b):
    br = 2000

    def body(t_ref, sp_ref, rp_ref, ws_ref, wr_ref, c_ref, s_ref, b_ref,
             o_ref, nsum_ref, esum_ref):
        i = pl.program_id(0)
        sent_blk = sp_ref[...]
        recv_blk = rp_ref[...]
        x = (t_ref[...]
             + jnp.dot(sent_blk, ws_ref[...], preferred_element_type=jnp.float32)
             + jnp.dot(recv_blk, wr_ref[...], preferred_element_type=jnp.float32)
             + c_ref[...])
        h = _ln_relu(x, s_ref[...], b_ref[...])
        o_ref[...] = h

        @pl.when(i == 0)
        def _():
            nsum_ref[...] = jnp.zeros_like(nsum_ref)
            esum_ref[...] = jnp.zeros_like(esum_ref)

        nsum_ref[...] += jnp.sum(h, axis=0, keepdims=True)
        esum_ref[...] += jnp.sum(sent_blk, axis=0, keepdims=True)

    one = jax.ShapeDtypeStruct((1, D), jnp.float32)
    return pl.pallas_call(
        body,
        grid=(N // br,),
        in_specs=[
            pl.BlockSpec((br, D), lambda i: (i, 0)),
            pl.BlockSpec((br, D), lambda i: (i, 0)),
            pl.BlockSpec((br, D), lambda i: (i, 0)),
            pl.BlockSpec((D, D), lambda i: (0, 0)),
            pl.BlockSpec((D, D), lambda i: (0, 0)),
            pl.BlockSpec((1, D), lambda i: (0, 0)),
            pl.BlockSpec((1, D), lambda i: (0, 0)),
            pl.BlockSpec((1, D), lambda i: (0, 0)),
        ],
        out_specs=[
            pl.BlockSpec((br, D), lambda i: (i, 0)),
            pl.BlockSpec((1, D), lambda i: (0, 0)),
            pl.BlockSpec((1, D), lambda i: (0, 0)),
        ],
        out_shape=[jax.ShapeDtypeStruct((N, D), jnp.float32), one, one],
    )(t, sent, recv, w_s, w_r, c, s, b)


def _global_update(nsum, esum, g, wg, bg, lns, lnb, w_e_g, be, w_n_g, bn):
    def body(ns_ref, es_ref, g_ref, wg_ref, bg_ref, s_ref, b_ref,
             weg_ref, be_ref, wng_ref, bn_ref, g_out, ce_out, cn_out):
        wg = wg_ref[...]
        x = (jnp.dot(ns_ref[...], wg[0:D, :], preferred_element_type=jnp.float32)
             + jnp.dot(es_ref[...], wg[D:2 * D, :], preferred_element_type=jnp.float32)
             + jnp.dot(g_ref[...], wg[2 * D:3 * D, :], preferred_element_type=jnp.float32)
             + bg_ref[...])
        gn = _ln_relu(x, s_ref[...], b_ref[...])
        g_out[...] = gn
        ce_out[...] = jnp.dot(gn, weg_ref[...], preferred_element_type=jnp.float32) + be_ref[...]
        cn_out[...] = jnp.dot(gn, wng_ref[...], preferred_element_type=jnp.float32) + bn_ref[...]

    one = jax.ShapeDtypeStruct((1, D), jnp.float32)
    return pl.pallas_call(
        body,
        out_shape=[one, one, one],
    )(nsum, esum, g, wg, bg, lns, lnb, w_e_g, be, w_n_g, bn)


def _global_final(nsum, esum, g, wg, bg, lns, lnb, wdec, bdec):
    def body(ns_ref, es_ref, g_ref, wg_ref, bg_ref, s_ref, b_ref,
             wd_ref, bd_ref, o_ref):
        wg = wg_ref[...]
        x = (jnp.dot(ns_ref[...], wg[0:D, :], preferred_element_type=jnp.float32)
             + jnp.dot(es_ref[...], wg[D:2 * D, :], preferred_element_type=jnp.float32)
             + jnp.dot(g_ref[...], wg[2 * D:3 * D, :], preferred_element_type=jnp.float32)
             + bg_ref[...])
        gn = _ln_relu(x, s_ref[...], b_ref[...])
        o_ref[...] = jnp.dot(gn, wd_ref[...], preferred_element_type=jnp.float32) + bd_ref[...]

    return pl.pallas_call(
        body,
        out_shape=jax.ShapeDtypeStruct((1, D), jnp.float32),
    )(nsum, esum, g, wg, bg, lns, lnb, wdec, bdec)


def _rsqrt16(x):
    i = lax.bitcast_convert_type(x, jnp.int32)
    i = jnp.int32(0x5F3759DF) - lax.shift_right_arithmetic(i, jnp.int32(1))
    y = lax.bitcast_convert_type(i, jnp.float32)
    for _ in range(3):
        y = y * (1.5 - 0.5 * x * y * y)
    return y


def _sc_edge(m_arr, table_a, table_b, senders, receivers, lns, lnb):
    nchunks = EPT // CHUNK
    buf2 = lambda shape, dt: [pltpu.VMEM(shape, dt), pltpu.VMEM(shape, dt)]

    @functools.partial(
        pl.kernel,
        out_type=jax.ShapeDtypeStruct((E, D), jnp.float32),
        mesh=_MESH,
        scratch_types=(
            buf2((CHUNK,), jnp.int32) + buf2((CHUNK,), jnp.int32)
            + buf2((CHUNK, D), jnp.float32) + buf2((CHUNK, D), jnp.float32)
            + buf2((CHUNK, D), jnp.float32) + buf2((CHUNK, D), jnp.float32)
            + [pltpu.VMEM((D,), jnp.float32), pltpu.VMEM((D,), jnp.float32)]
            + [pltpu.SemaphoreType.DMA] * 12
        ),
    )
    def k(m_hbm, ta_hbm, tb_hbm, s_hbm, r_hbm, lns_hbm, lnb_hbm, out_hbm,
          ia0, ia1, ib0, ib1, m0, m1, a0, a1, b0, b1, o0, o1,
          lns_v, lnb_v,
          sia0, sia1, sib0, sib1, sm0, sm1, sa0, sa1, sb0, sb1, so0, so1):
        wid = lax.axis_index("s") * NC + lax.axis_index("c")
        base = wid * EPT
        B = [dict(ia=ia0, ib=ib0, m=m0, a=a0, b=b0, o=o0, sia=sia0, sib=sib0,
                  sm=sm0, sa=sa0, sb=sb0, so=so0),
             dict(ia=ia1, ib=ib1, m=m1, a=a1, b=b1, o=o1, sia=sia1, sib=sib1,
                  sm=sm1, sa=sa1, sb=sb1, so=so1)]

        pltpu.sync_copy(lns_hbm, lns_v)
        pltpu.sync_copy(lnb_hbm, lnb_v)
        sregs = [lns_v[pl.ds(16 * j, 16)] for j in range(8)]
        bregs = [lnb_v[pl.ds(16 * j, 16)] for j in range(8)]

        def offs(c):
            return base + c * CHUNK

        def issue_idx(c, P):
            pltpu.async_copy(s_hbm.at[pl.ds(offs(c), CHUNK)], P['ia'], P['sia'])
            pltpu.async_copy(r_hbm.at[pl.ds(offs(c), CHUNK)], P['ib'], P['sib'])

        def wait_idx(c, P):
            pltpu.make_async_copy(s_hbm.at[pl.ds(offs(c), CHUNK)], P['ia'], P['sia']).wait()
            pltpu.make_async_copy(r_hbm.at[pl.ds(offs(c), CHUNK)], P['ib'], P['sib']).wait()

        def issue_main(c, P):
            pltpu.async_copy(m_hbm.at[pl.ds(offs(c), CHUNK)], P['m'], P['sm'])
            pltpu.async_copy(ta_hbm.at[P['ia']], P['a'], P['sa'])
            pltpu.async_copy(tb_hbm.at[P['ib']], P['b'], P['sb'])

        def wait_main(c, P):
            pltpu.make_async_copy(m_hbm.at[pl.ds(offs(c), CHUNK)], P['m'], P['sm']).wait()
            pltpu.make_async_copy(ta_hbm.at[P['ia']], P['a'], P['sa']).wait()
            pltpu.make_async_copy(tb_hbm.at[P['ib']], P['b'], P['sb']).wait()

        def issue_out(c, P):
            pltpu.async_copy(P['o'], out_hbm.at[pl.ds(offs(c), CHUNK)], P['so'])

        def wait_out(c, P):
            pltpu.make_async_copy(P['o'], out_hbm.at[pl.ds(offs(c), CHUNK)], P['so']).wait()

        def compute(P):
            m_v, a_v, b_v, o_v = P['m'], P['a'], P['b'], P['o']

            @plsc.parallel_loop(0, CHUNK, unroll=4)
            def row(r):
                xs = [m_v[r, pl.ds(16 * j, 16)] + a_v[r, pl.ds(16 * j, 16)]
                      + b_v[r, pl.ds(16 * j, 16)] for j in range(8)]
                sv = (((xs[0] + xs[1]) + (xs[2] + xs[3]))
                      + ((xs[4] + xs[5]) + (xs[6] + xs[7])))
                qs = [x * x for x in xs]
                qv = (((qs[0] + qs[1]) + (qs[2] + qs[3]))
                      + ((qs[4] + qs[5]) + (qs[6] + qs[7])))

                def lanesum(v):
                    p = [v[2 * t] + v[2 * t + 1] for t in range(8)]
                    p = [p[2 * t] + p[2 * t + 1] for t in range(4)]
                    p = [p[2 * t] + p[2 * t + 1] for t in range(2)]
                    return p[0] + p[1]

                mean = lanesum(sv) * (1.0 / D)
                var = lanesum(qv) * (1.0 / D) - mean * mean
                rsv = _rsqrt16(jnp.full((16,), var + 1e-6, jnp.float32))
                mv = jnp.full((16,), mean, jnp.float32)
                for j in range(8):
                    y = (xs[j] - mv) * (rsv * sregs[j]) + bregs[j]
                    o_v[r, pl.ds(16 * j, 16)] = jnp.maximum(y, 0.0)

        issue_idx(0, B[0])
        wait_idx(0, B[0])
        issue_main(0, B[0])
        issue_idx(1, B[1])

        def body(kk, carry):
            for bsel in (0, 1):
                c = 2 * kk + bsel
                P, Q = B[bsel], B[1 - bsel]
                wait_idx(c + 1, Q)
                issue_main(c + 1, Q)
                wait_main(c, P)

                @pl.when(c + 2 < nchunks)
                def _():
                    issue_idx(c + 2, P)

                @pl.when(c >= 2)
                def _():
                    wait_out(c - 2, P)

                compute(P)
                issue_out(c, P)
            return carry

        lax.fori_loop(0, (nchunks - 1) // 2, body, 0)

        c_last = nchunks - 1
        wait_main(c_last, B[0])
        wait_out(c_last - 2, B[0])
        compute(B[0])
        issue_out(c_last, B[0])
        wait_out(c_last - 1, B[1])
        wait_out(c_last, B[0])

    return k(m_arr, table_a, table_b, senders, receivers, lns, lnb)


def _sc_segsum2(data, senders, receivers):
    @functools.partial(
        pl.kernel,
        out_type=[jax.ShapeDtypeStruct((NPAD, D), jnp.float32),
                  jax.ShapeDtypeStruct((NPAD, D), jnp.float32)],
        mesh=_MESH,
        scratch_types=[
            pltpu.VMEM((CHUNK,), jnp.int32),
            pltpu.VMEM((CHUNK,), jnp.int32),
            pltpu.VMEM((CHUNK, D), jnp.float32),
            pltpu.VMEM((CHUNK, D), jnp.float32),
            pltpu.VMEM((CHUNK, D), jnp.float32),
            pltpu.VMEM_SHARED((NPAD, D), jnp.float32),
            pltpu.SemaphoreType.DMA,
            pltpu.SemaphoreType.DMA,
            pltpu.SemaphoreType.DMA,
            pltpu.SemaphoreType.DMA,
            pltpu.SemaphoreType.DMA,
            pltpu.SemaphoreType.DMA,
        ],
    )
    def k(d_hbm, s_hbm, r_hbm, sent_hbm, recv_hbm,
          i0, i1, r0, r1, zbuf, acc, si0, si1, sr0, sr1, ss0, ss1):
        core = lax.axis_index("c")
        sid = lax.axis_index("s")
        B = [dict(i=i0, r=r0, si=si0, sr=sr0, ss=ss0),
             dict(i=i1, r=r1, si=si1, sr=sr1, ss=ss1)]

        def zbody(kk, carry):
            i = kk // 8
            j = (kk % 8) * 16
            zbuf[i, pl.ds(j, 16)] = jnp.zeros((16,), jnp.float32)
            return carry

        lax.fori_loop(0, CHUNK * 8, zbody, 0)
        for t in range(STRIPE // CHUNK):
            pltpu.sync_copy(zbuf, acc.at[pl.ds(sid * STRIPE + t * CHUNK, CHUNK)])
        plsc.subcore_barrier()

        nchunks = EPS // CHUNK

        def run(idx_hbm):
            def offs(c):
                return sid * EPS + c * CHUNK

            def issue_loads(c, P):
                pltpu.async_copy(idx_hbm.at[pl.ds(offs(c), CHUNK)], P['i'], P['si'])
                pltpu.async_copy(d_hbm.at[pl.ds(offs(c), CHUNK)], P['r'], P['sr'])

            def wait_loads(c, P):
                pltpu.make_async_copy(idx_hbm.at[pl.ds(offs(c), CHUNK)], P['i'], P['si']).wait()
                pltpu.make_async_copy(d_hbm.at[pl.ds(offs(c), CHUNK)], P['r'], P['sr']).wait()

            def wait_scat(P):
                pltpu.make_async_copy(P['r'], acc.at[P['i']], P['ss']).wait()

            issue_loads(0, B[0])

            def body(kk, carry):
                for bsel in (0, 1):
                    c = 2 * kk + bsel
                    P, Q = B[bsel], B[1 - bsel]
                    wait_loads(c, P)
                    pltpu.async_copy(P['r'], acc.at[P['i']], P['ss'], add=True)

                    @pl.when(c + 1 < nchunks)
                    def _():
                        @pl.when(c >= 1)
                        def _():
                            wait_scat(Q)
                        issue_loads(c + 1, Q)
                return carry

            lax.fori_loop(0, nchunks // 2, body, 0)
            wait_scat(B[0])
            wait_scat(B[1])

        @pl.when(core == 0)
        def _():
            run(s_hbm)

        @pl.when(core == 1)
        def _():
            run(r_hbm)

        plsc.subcore_barrier()

        @pl.when(core == 0)
        def _():
            pltpu.sync_copy(acc.at[pl.ds(sid * STRIPE, STRIPE)],
                            sent_hbm.at[pl.ds(sid * STRIPE, STRIPE)])

        @pl.when(core == 1)
        def _():
            pltpu.sync_copy(acc.at[pl.ds(sid * STRIPE, STRIPE)],
                            recv_hbm.at[pl.ds(sid * STRIPE, STRIPE)])

    return k(data, senders, receivers)


def kernel(nodes, edge_attr, senders, receivers, train, params):
    del train
    senders = senders.astype(jnp.int32)
    receivers = receivers.astype(jnp.int32)

    h_n = _mm(nodes, params['en']['W'], params['en']['b'][None])
    h_e = _mm(edge_attr, params['ee']['W'], params['ee']['b'][None])
    g = jnp.zeros((1, D), jnp.float32)

    c_e = params['steps'][0]['e']['b'][None]
    c_n = params['steps'][0]['n']['b'][None]

    out = None
    for i, sp in enumerate(params['steps']):
        we = sp['e']['W']
        wn = sp['n']['W']

        a_tab, b_tab, t_arr = _node_pre(h_n, we[D:2 * D], we[2 * D:3 * D],
                                        wn[0:D])
        m_arr = _mm(h_e, we[0:D], c_e)
        h_e = _sc_edge(m_arr, a_tab, b_tab, senders, receivers,
                       sp['e']['ln_s'], sp['e']['ln_b'])
        sent, recv = _sc_segsum2(h_e, senders, receivers)
        h_n, nsum, esum = _node_update(
            t_arr, sent[:N], recv[:N], wn[D:2 * D], wn[2 * D:3 * D],
            c_n, sp['n']['ln_s'][None], sp['n']['ln_b'][None])

        gp = sp['g']
        if i + 1 < len(params['steps']):
            nxt = params['steps'][i + 1]
            g, c_e, c_n = _global_update(
                nsum, esum, g, gp['W'], gp['b'][None],
                gp['ln_s'][None], gp['ln_b'][None],
                nxt['e']['W'][3 * D:], nxt['e']['b'][None],
                nxt['n']['W'][3 * D:], nxt['n']['b'][None])
        else:
            out = _global_final(
                nsum, esum, g, gp['W'], gp['b'][None],
                gp['ln_s'][None], gp['ln_b'][None],
                params['dec']['W'], params['dec']['b'][None])

    return out


# speedup vs baseline: 2.1121x; 1.0345x over previous
import functools

import jax
import jax.numpy as jnp
from jax import lax
from jax.experimental import pallas as pl
from jax.experimental.pallas import tpu as pltpu
from jax.experimental.pallas import tpu_sc as plsc

N = 10000
E = 320000
D = 128

NC = 2
NS = 16
NW = NC * NS

NPAD = 10240
STRIPE = NPAD // NS

CHUNK = 80
EPT = E // NW
EPS = E // NS

_MESH = plsc.VectorSubcoreMesh(
    core_axis_name="c", subcore_axis_name="s", num_cores=NC, num_subcores=NS)


def _mm(x, w, c, br=2000):
    r, k = x.shape
    dout = w.shape[1]

    def body(x_ref, w_ref, c_ref, o_ref):
        o_ref[...] = (
            jnp.dot(x_ref[...], w_ref[...], preferred_element_type=jnp.float32)
            + c_ref[...])

    return pl.pallas_call(
        body,
        grid=(r // br,),
        in_specs=[
            pl.BlockSpec((br, k), lambda i: (i, 0)),
            pl.BlockSpec((k, dout), lambda i: (0, 0)),
            pl.BlockSpec((1, dout), lambda i: (0, 0)),
        ],
        out_specs=pl.BlockSpec((br, dout), lambda i: (i, 0)),
        out_shape=jax.ShapeDtypeStruct((r, dout), jnp.float32),
    )(x, w, c)


def _mm2(x, w1, c1, w2, c2, br=2000):
    r, k = x.shape
    dout = w2.shape[1]

    def body(x_ref, w1_ref, c1_ref, w2_ref, c2_ref, o_ref):
        h = (jnp.dot(x_ref[...], w1_ref[...], preferred_element_type=jnp.float32)
             + c1_ref[...])
        o_ref[...] = (
            jnp.dot(h, w2_ref[...], preferred_element_type=jnp.float32)
            + c2_ref[...])

    return pl.pallas_call(
        body,
        grid=(r // br,),
        in_specs=[
            pl.BlockSpec((br, k), lambda i: (i, 0)),
            pl.BlockSpec((k, w1.shape[1]), lambda i: (0, 0)),
            pl.BlockSpec((1, w1.shape[1]), lambda i: (0, 0)),
            pl.BlockSpec((w2.shape[0], dout), lambda i: (0, 0)),
            pl.BlockSpec((1, dout), lambda i: (0, 0)),
        ],
        out_specs=pl.BlockSpec((br, dout), lambda i: (i, 0)),
        out_shape=jax.ShapeDtypeStruct((r, dout), jnp.float32),
    )(x, w1, c1, w2, c2)


def _ln_relu(x, s, b):
    m = jnp.mean(x, axis=-1, keepdims=True)
    xc = x - m
    v = jnp.mean(xc * xc, axis=-1, keepdims=True)
    return jax.nn.relu(xc * lax.rsqrt(v + 1e-6) * s + b)


def _edge_finish(m_arr, ga, gb, s, b, br=2000):
    def body(m_ref, a_ref, b_ref, s_ref, bb_ref, o_ref):
        x = m_ref[...] + a_ref[...] + b_ref[...]
        o_ref[...] = _ln_relu(x, s_ref[...], bb_ref[...])

    return pl.pallas_call(
        body,
        grid=(E // br,),
        in_specs=[
            pl.BlockSpec((br, D), lambda i: (i, 0)),
            pl.BlockSpec((br, D), lambda i: (i, 0)),
            pl.BlockSpec((br, D), lambda i: (i, 0)),
            pl.BlockSpec((1, D), lambda i: (0, 0)),
            pl.BlockSpec((1, D), lambda i: (0, 0)),
        ],
        out_specs=pl.BlockSpec((br, D), lambda i: (i, 0)),
        out_shape=jax.ShapeDtypeStruct((E, D), jnp.float32),
    )(m_arr, ga, gb, s, b)


def _node_pre(h_n, w_a, w_b, w_t):
    def body(x_ref, wa_ref, wb_ref, wt_ref, a_ref, b_ref, t_ref):
        x = x_ref[...]
        a_ref[...] = jnp.dot(x, wa_ref[...], preferred_element_type=jnp.float32)
        b_ref[...] = jnp.dot(x, wb_ref[...], preferred_element_type=jnp.float32)
        t_ref[...] = jnp.dot(x, wt_ref[...], preferred_element_type=jnp.float32)

    br = 2000
    sds = jax.ShapeDtypeStruct((N, D), jnp.float32)
    return pl.pallas_call(
        body,
        grid=(N // br,),
        in_specs=[
            pl.BlockSpec((br, D), lambda i: (i, 0)),
            pl.BlockSpec((D, D), lambda i: (0, 0)),
            pl.BlockSpec((D, D), lambda i: (0, 0)),
            pl.BlockSpec((D, D), lambda i: (0, 0)),
        ],
        out_specs=[
            pl.BlockSpec((br, D), lambda i: (i, 0)),
            pl.BlockSpec((br, D), lambda i: (i, 0)),
            pl.BlockSpec((br, D), lambda i: (i, 0)),
        ],
        out_shape=[sds, sds, sds],
    )(h_n, w_a, w_b, w_t)


def _node_update(t, sent, recv, w_s, w_r, c, s, b):
    br = 2000

    def body(t_ref, sp_ref, rp_ref, ws_ref, wr_ref, c_ref, s_ref, b_ref,
             o_ref, nsum_ref, esum_ref):
        i = pl.program_id(0)
        sent_blk = sp_ref[...]
        recv_blk = rp_ref[...]
        x = (t_ref[...]
             + jnp.dot(sent_blk, ws_ref[...], preferred_element_type=jnp.float32)
             + jnp.dot(recv_blk, wr_ref[...], preferred_element_type=jnp.float32)
             + c_ref[...])
        h = _ln_relu(x, s_ref[...], b_ref[...])
        o_ref[...] = h

        @pl.when(i == 0)
        def _():
            nsum_ref[...] = jnp.zeros_like(nsum_ref)
            esum_ref[...] = jnp.zeros_like(esum_ref)

        nsum_ref[...] += jnp.sum(h, axis=0, keepdims=True)
        esum_ref[...] += jnp.sum(sent_blk, axis=0, keepdims=True)

    one = jax.ShapeDtypeStruct((1, D), jnp.float32)
    return pl.pallas_call(
        body,
        grid=(N // br,),
        in_specs=[
            pl.BlockSpec((br, D), lambda i: (i, 0)),
            pl.BlockSpec((br, D), lambda i: (i, 0)),
            pl.BlockSpec((br, D), lambda i: (i, 0)),
            pl.BlockSpec((D, D), lambda i: (0, 0)),
            pl.BlockSpec((D, D), lambda i: (0, 0)),
            pl.BlockSpec((1, D), lambda i: (0, 0)),
            pl.BlockSpec((1, D), lambda i: (0, 0)),
            pl.BlockSpec((1, D), lambda i: (0, 0)),
        ],
        out_specs=[
            pl.BlockSpec((br, D), lambda i: (i, 0)),
            pl.BlockSpec((1, D), lambda i: (0, 0)),
            pl.BlockSpec((1, D), lambda i: (0, 0)),
        ],
        out_shape=[jax.ShapeDtypeStruct((N, D), jnp.float32), one, one],
    )(t, sent, recv, w_s, w_r, c, s, b)


def _global_update(nsum, esum, g, wg, bg, lns, lnb, w_e_g, be, w_n_g, bn):
    def body(ns_ref, es_ref, g_ref, wg_ref, bg_ref, s_ref, b_ref,
             weg_ref, be_ref, wng_ref, bn_ref, g_out, ce_out, cn_out):
        wg = wg_ref[...]
        x = (jnp.dot(ns_ref[...], wg[0:D, :], preferred_element_type=jnp.float32)
             + jnp.dot(es_ref[...], wg[D:2 * D, :], preferred_element_type=jnp.float32)
             + jnp.dot(g_ref[...], wg[2 * D:3 * D, :], preferred_element_type=jnp.float32)
             + bg_ref[...])
        gn = _ln_relu(x, s_ref[...], b_ref[...])
        g_out[...] = gn
        ce_out[...] = jnp.dot(gn, weg_ref[...], preferred_element_type=jnp.float32) + be_ref[...]
        cn_out[...] = jnp.dot(gn, wng_ref[...], preferred_element_type=jnp.float32) + bn_ref[...]

    one = jax.ShapeDtypeStruct((1, D), jnp.float32)
    return pl.pallas_call(
        body,
        out_shape=[one, one, one],
    )(nsum, esum, g, wg, bg, lns, lnb, w_e_g, be, w_n_g, bn)


def _global_final(nsum, esum, g, wg, bg, lns, lnb, wdec, bdec):
    def body(ns_ref, es_ref, g_ref, wg_ref, bg_ref, s_ref, b_ref,
             wd_ref, bd_ref, o_ref):
        wg = wg_ref[...]
        x = (jnp.dot(ns_ref[...], wg[0:D, :], preferred_element_type=jnp.float32)
             + jnp.dot(es_ref[...], wg[D:2 * D, :], preferred_element_type=jnp.float32)
             + jnp.dot(g_ref[...], wg[2 * D:3 * D, :], preferred_element_type=jnp.float32)
             + bg_ref[...])
        gn = _ln_relu(x, s_ref[...], b_ref[...])
        o_ref[...] = jnp.dot(gn, wd_ref[...], preferred_element_type=jnp.float32) + bd_ref[...]

    return pl.pallas_call(
        body,
        out_shape=jax.ShapeDtypeStruct((1, D), jnp.float32),
    )(nsum, esum, g, wg, bg, lns, lnb, wdec, bdec)


def _rsqrt16(x):
    i = lax.bitcast_convert_type(x, jnp.int32)
    i = jnp.int32(0x5F3759DF) - lax.shift_right_arithmetic(i, jnp.int32(1))
    y = lax.bitcast_convert_type(i, jnp.float32)
    for _ in range(3):
        y = y * (1.5 - 0.5 * x * y * y)
    return y


def _sc_edge(m_arr, table_a, table_b, senders, receivers, lns, lnb):
    nchunks = EPT // CHUNK
    buf2 = lambda shape, dt: [pltpu.VMEM(shape, dt), pltpu.VMEM(shape, dt)]

    @functools.partial(
        pl.kernel,
        out_type=jax.ShapeDtypeStruct((E, D), jnp.float32),
        mesh=_MESH,
        scratch_types=(
            buf2((CHUNK,), jnp.int32) + buf2((CHUNK,), jnp.int32)
            + buf2((CHUNK, D), jnp.float32) + buf2((CHUNK, D), jnp.float32)
            + buf2((CHUNK, D), jnp.float32) + buf2((CHUNK, D), jnp.float32)
            + [pltpu.VMEM((D,), jnp.float32), pltpu.VMEM((D,), jnp.float32)]
            + [pltpu.SemaphoreType.DMA] * 12
        ),
    )
    def k(m_hbm, ta_hbm, tb_hbm, s_hbm, r_hbm, lns_hbm, lnb_hbm, out_hbm,
          ia0, ia1, ib0, ib1, m0, m1, a0, a1, b0, b1, o0, o1,
          lns_v, lnb_v,
          sia0, sia1, sib0, sib1, sm0, sm1, sa0, sa1, sb0, sb1, so0, so1):
        wid = lax.axis_index("s") * NC + lax.axis_index("c")
        base = wid * EPT
        B = [dict(ia=ia0, ib=ib0, m=m0, a=a0, b=b0, o=o0, sia=sia0, sib=sib0,
                  sm=sm0, sa=sa0, sb=sb0, so=so0),
             dict(ia=ia1, ib=ib1, m=m1, a=a1, b=b1, o=o1, sia=sia1, sib=sib1,
                  sm=sm1, sa=sa1, sb=sb1, so=so1)]

        pltpu.sync_copy(lns_hbm, lns_v)
        pltpu.sync_copy(lnb_hbm, lnb_v)
        sregs = [lns_v[pl.ds(16 * j, 16)] for j in range(8)]
        bregs = [lnb_v[pl.ds(16 * j, 16)] for j in range(8)]

        def offs(c):
            return base + c * CHUNK

        def issue_idx(c, P):
            pltpu.async_copy(s_hbm.at[pl.ds(offs(c), CHUNK)], P['ia'], P['sia'])
            pltpu.async_copy(r_hbm.at[pl.ds(offs(c), CHUNK)], P['ib'], P['sib'])

        def wait_idx(c, P):
            pltpu.make_async_copy(s_hbm.at[pl.ds(offs(c), CHUNK)], P['ia'], P['sia']).wait()
            pltpu.make_async_copy(r_hbm.at[pl.ds(offs(c), CHUNK)], P['ib'], P['sib']).wait()

        def issue_main(c, P):
            pltpu.async_copy(m_hbm.at[pl.ds(offs(c), CHUNK)], P['m'], P['sm'])
            pltpu.async_copy(ta_hbm.at[P['ia']], P['a'], P['sa'])
            pltpu.async_copy(tb_hbm.at[P['ib']], P['b'], P['sb'])

        def wait_main(c, P):
            pltpu.make_async_copy(m_hbm.at[pl.ds(offs(c), CHUNK)], P['m'], P['sm']).wait()
            pltpu.make_async_copy(ta_hbm.at[P['ia']], P['a'], P['sa']).wait()
            pltpu.make_async_copy(tb_hbm.at[P['ib']], P['b'], P['sb']).wait()

        def issue_out(c, P):
            pltpu.async_copy(P['o'], out_hbm.at[pl.ds(offs(c), CHUNK)], P['so'])

        def wait_out(c, P):
            pltpu.make_async_copy(P['o'], out_hbm.at[pl.ds(offs(c), CHUNK)], P['so']).wait()

        def compute(P):
            m_v, a_v, b_v, o_v = P['m'], P['a'], P['b'], P['o']

            @plsc.parallel_loop(0, CHUNK, unroll=4)
            def row(r):
                xs = [m_v[r, pl.ds(16 * j, 16)] + a_v[r, pl.ds(16 * j, 16)]
                      + b_v[r, pl.ds(16 * j, 16)] for j in range(8)]
                sv = (((xs[0] + xs[1]) + (xs[2] + xs[3]))
                      + ((xs[4] + xs[5]) + (xs[6] + xs[7])))
                qs = [x * x for x in xs]
                qv = (((qs[0] + qs[1]) + (qs[2] + qs[3]))
                      + ((qs[4] + qs[5]) + (qs[6] + qs[7])))

                def lanesum(v):
                    p = [v[2 * t] + v[2 * t + 1] for t in range(8)]
                    p = [p[2 * t] + p[2 * t + 1] for t in range(4)]
                    p = [p[2 * t] + p[2 * t + 1] for t in range(2)]
                    return p[0] + p[1]

                mean = lanesum(sv) * (1.0 / D)
                var = lanesum(qv) * (1.0 / D) - mean * mean
                rsv = _rsqrt16(jnp.full((16,), var + 1e-6, jnp.float32))
                mv = jnp.full((16,), mean, jnp.float32)
                for j in range(8):
                    y = (xs[j] - mv) * (rsv * sregs[j]) + bregs[j]
                    o_v[r, pl.ds(16 * j, 16)] = jnp.maximum(y, 0.0)

        issue_idx(0, B[0])
        wait_idx(0, B[0])
        issue_main(0, B[0])
        issue_idx(1, B[1])

        def body(kk, carry):
            for bsel in (0, 1):
                c = 2 * kk + bsel
                P, Q = B[bsel], B[1 - bsel]
                wait_idx(c + 1, Q)
                issue_main(c + 1, Q)
                wait_main(c, P)

                @pl.when(c + 2 < nchunks)
                def _():
                    issue_idx(c + 2, P)

                @pl.when(c >= 2)
                def _():
                    wait_out(c - 2, P)

                compute(P)
                issue_out(c, P)
            return carry

        lax.fori_loop(0, (nchunks - 1) // 2, body, 0)

        c_last = nchunks - 1
        wait_main(c_last, B[0])
        wait_out(c_last - 2, B[0])
        compute(B[0])
        issue_out(c_last, B[0])
        wait_out(c_last - 1, B[1])
        wait_out(c_last, B[0])

    return k(m_arr, table_a, table_b, senders, receivers, lns, lnb)


def _sc_segsum2(data, senders, receivers):
    @functools.partial(
        pl.kernel,
        out_type=[jax.ShapeDtypeStruct((NPAD, D), jnp.float32),
                  jax.ShapeDtypeStruct((NPAD, D), jnp.float32)],
        mesh=_MESH,
        scratch_types=[
            pltpu.VMEM((CHUNK,), jnp.int32),
            pltpu.VMEM((CHUNK,), jnp.int32),
            pltpu.VMEM((CHUNK, D), jnp.float32),
            pltpu.VMEM((CHUNK, D), jnp.float32),
            pltpu.VMEM((CHUNK, D), jnp.float32),
            pltpu.VMEM_SHARED((NPAD, D), jnp.float32),
            pltpu.SemaphoreType.DMA,
            pltpu.SemaphoreType.DMA,
            pltpu.SemaphoreType.DMA,
            pltpu.SemaphoreType.DMA,
            pltpu.SemaphoreType.DMA,
            pltpu.SemaphoreType.DMA,
        ],
    )
    def k(d_hbm, s_hbm, r_hbm, sent_hbm, recv_hbm,
          i0, i1, r0, r1, zbuf, acc, si0, si1, sr0, sr1, ss0, ss1):
        core = lax.axis_index("c")
        sid = lax.axis_index("s")
        B = [dict(i=i0, r=r0, si=si0, sr=sr0, ss=ss0),
             dict(i=i1, r=r1, si=si1, sr=sr1, ss=ss1)]

        def zbody(kk, carry):
            i = kk // 8
            j = (kk % 8) * 16
            zbuf[i, pl.ds(j, 16)] = jnp.zeros((16,), jnp.float32)
            return carry

        lax.fori_loop(0, CHUNK * 8, zbody, 0)
        for t in range(STRIPE // CHUNK):
            pltpu.sync_copy(zbuf, acc.at[pl.ds(sid * STRIPE + t * CHUNK, CHUNK)])
        plsc.subcore_barrier()

        nchunks = EPS // CHUNK

        def run(idx_hbm):
            def offs(c):
                return sid * EPS + c * CHUNK

            def issue_loads(c, P):
                pltpu.async_copy(idx_hbm.at[pl.ds(offs(c), CHUNK)], P['i'], P['si'])
                pltpu.async_copy(d_hbm.at[pl.ds(offs(c), CHUNK)], P['r'], P['sr'])

            def wait_loads(c, P):
                pltpu.make_async_copy(idx_hbm.at[pl.ds(offs(c), CHUNK)], P['i'], P['si']).wait()
                pltpu.make_async_copy(d_hbm.at[pl.ds(offs(c), CHUNK)], P['r'], P['sr']).wait()

            def wait_scat(P):
                pltpu.make_async_copy(P['r'], acc.at[P['i']], P['ss']).wait()

            issue_loads(0, B[0])

            def body(kk, carry):
                for bsel in (0, 1):
                    c = 2 * kk + bsel
                    P, Q = B[bsel], B[1 - bsel]
                    wait_loads(c, P)
                    pltpu.async_copy(P['r'], acc.at[P['i']], P['ss'], add=True)

                    @pl.when(c + 1 < nchunks)
                    def _():
                        @pl.when(c >= 1)
                        def _():
                            wait_scat(Q)
                        issue_loads(c + 1, Q)
                return carry

            lax.fori_loop(0, nchunks // 2, body, 0)
            wait_scat(B[0])
            wait_scat(B[1])

        @pl.when(core == 0)
        def _():
            run(s_hbm)

        @pl.when(core == 1)
        def _():
            run(r_hbm)

        plsc.subcore_barrier()

        @pl.when(core == 0)
        def _():
            pltpu.sync_copy(acc.at[pl.ds(sid * STRIPE, STRIPE)],
                            sent_hbm.at[pl.ds(sid * STRIPE, STRIPE)])

        @pl.when(core == 1)
        def _():
            pltpu.sync_copy(acc.at[pl.ds(sid * STRIPE, STRIPE)],
                            recv_hbm.at[pl.ds(sid * STRIPE, STRIPE)])

    return k(data, senders, receivers)


def kernel(nodes, edge_attr, senders, receivers, train, params):
    del train
    senders = senders.astype(jnp.int32)
    receivers = receivers.astype(jnp.int32)

    h_n = _mm(nodes, params['en']['W'], params['en']['b'][None])
    h_e = None
    g = jnp.zeros((1, D), jnp.float32)

    c_e = params['steps'][0]['e']['b'][None]
    c_n = params['steps'][0]['n']['b'][None]

    out = None
    for i, sp in enumerate(params['steps']):
        we = sp['e']['W']
        wn = sp['n']['W']

        a_tab, b_tab, t_arr = _node_pre(h_n, we[D:2 * D], we[2 * D:3 * D],
                                        wn[0:D])
        if i == 0:
            m_arr = _mm2(edge_attr, params['ee']['W'],
                         params['ee']['b'][None], we[0:D], c_e)
        else:
            m_arr = _mm(h_e, we[0:D], c_e)
        h_e = _sc_edge(m_arr, a_tab, b_tab, senders, receivers,
                       sp['e']['ln_s'], sp['e']['ln_b'])
        sent, recv = _sc_segsum2(h_e, senders, receivers)
        h_n, nsum, esum = _node_update(
            t_arr, sent[:N], recv[:N], wn[D:2 * D], wn[2 * D:3 * D],
            c_n, sp['n']['ln_s'][None], sp['n']['ln_b'][None])

        gp = sp['g']
        if i + 1 < len(params['steps']):
            nxt = params['steps'][i + 1]
            g, c_e, c_n = _global_update(
                nsum, esum, g, gp['W'], gp['b'][None],
                gp['ln_s'][None], gp['ln_b'][None],
                nxt['e']['W'][3 * D:], nxt['e']['b'][None],
                nxt['n']['W'][3 * D:], nxt['n']['b'][None])
        else:
            out = _global_final(
                nsum, esum, g, gp['W'], gp['b'][None],
                gp['ln_s'][None], gp['ln_b'][None],
                params['dec']['W'], params['dec']['b'][None])

    return out
